# baseline (speedup 1.0000x reference)
import jax
import jax.numpy as jnp
from jax import lax
from jax.experimental import pallas as pl
from jax.experimental.pallas import tpu as pltpu
from jax.experimental.pallas import tpu_sc as plsc

N_ROWS = 16384
N_COLS = 16384
NNZ = 16
BATCH = 64
LANES = 16

NC, NS = 2, 16
NW = NC * NS
ROWS_PER_W = N_ROWS // NW
C = 32
NCH = ROWS_PER_W // C
IPC = C * NNZ
GW = 128
NG = IPC // GW
IDXROWS_PER_W = ROWS_PER_W * NNZ // GW
RPW_P = ROWS_PER_W + 1

COLS_PER_TILE = N_COLS // NS
PA_W = 256
PA_P = PA_W + 1
PA_PASSES = COLS_PER_TILE // PA_W


def _sc_body(data, idx2, vals, bias, out, table,
             pa_src, pa_src2, pa_dst, pa_dst2,
             idx_a, idx_b, vals_a, vals_b, bias_v, g_a, g_b, obuf,
             gsem_a, gsem_b, isem_a, isem_b, osem, pasem_i, pasem_o):
    cid = lax.axis_index("c")
    sid = lax.axis_index("s")
    wid = sid * NC + cid
    row0_w = wid * ROWS_PER_W
    ir0_w = wid * IDXROWS_PER_W

    iota = lax.iota(jnp.int32, LANES)

    col0 = sid * COLS_PER_TILE
    pa_srcs = (pa_src, pa_src2)
    pa_dsts = (pa_dst, pa_dst2)

    def pa_in(pa, q):
        return pltpu.make_async_copy(
            data.at[:, pl.ds(col0 + pa * PA_W, PA_W)],
            pa_srcs[q].at[:, pl.ds(0, PA_W)], pasem_i)

    def pa_out(pa, q):
        return pltpu.make_async_copy(
            pa_dsts[q], table.at[pl.ds(col0 + pa * PA_W, PA_W)], pasem_o)

    pa_in(0, 0).start()
    for pa in range(PA_PASSES):
        q = pa % 2
        pa_in(pa, q).wait()
        if pa + 1 < PA_PASSES:
            pa_in(pa + 1, 1 - q).start()
        if pa >= 2:
            pa_out(pa - 2, q).wait()
        src = pa_srcs[q]
        dst = pa_dsts[q]

        @pl.loop(0, PA_W, unroll=8)
        def _col(c):
            cidx = jnp.full((LANES,), c, jnp.int32)
            quads = [
                plsc.load_gather(src, [iota + h * LANES, cidx])
                for h in range(BATCH // LANES)
            ]
            lo = plsc.pack(quads[0], quads[1],
                           format=plsc.PackFormat.INTERLEAVED)
            hi = plsc.pack(quads[2], quads[3],
                           format=plsc.PackFormat.INTERLEAVED)
            dst[c, pl.ds(0, 32)] = lo
            dst[c, pl.ds(32, 32)] = hi

        pa_out(pa, q).start()
    pa_out(PA_PASSES - 2, 0).wait()
    pa_out(PA_PASSES - 1, 1).wait()

    plsc.subcore_barrier()

    idx_bufs = (idx_a, idx_b)
    vals_bufs = (vals_a, vals_b)
    g_bufs = (g_a, g_b)
    gsems = (gsem_a, gsem_b)
    isems = (isem_a, isem_b)

    def idx_copy(g, p):
        return pltpu.make_async_copy(
            idx2.at[pl.ds(ir0_w + g * NG, NG)], idx_bufs[p], isems[p])

    def gather_copies(g, p):
        nnz0 = (row0_w + g * C) * NNZ
        cps = [
            pltpu.make_async_copy(table.at[idx_bufs[p].at[j]],
                                  g_bufs[p].at[pl.ds(j * GW, GW)], gsems[p])
            for j in range(NG)
        ]
        cps.append(pltpu.make_async_copy(vals.at[pl.ds(nnz0, IPC)],
                                         vals_bufs[p], gsems[p]))
        return cps

    pltpu.sync_copy(idx2.at[pl.ds(ir0_w, NG)], idx_bufs[0])
    for cp in gather_copies(0, 0):
        cp.start()
    idx_copy(1, 1).start()

    pltpu.sync_copy(bias.at[pl.ds(row0_w, ROWS_PER_W)], bias_v)

    @pl.loop(0, BATCH)
    def _binit(b):
        for h in range(ROWS_PER_W // LANES):
            obuf[b, pl.ds(h * LANES, LANES)] = bias_v[pl.ds(h * LANES, LANES)]

    ev_idx = [iota + c32 * 32 for c32 in range(BATCH // 32)]
    od_idx = [iota + 16 + c32 * 32 for c32 in range(BATCH // 32)]

    @pl.loop(0, NCH, step=2)
    def _pair(g0):
        for p in range(2):
            cur = g0 + p
            for cp in gather_copies(cur, p):
                cp.wait()
            @pl.when(cur + 1 < NCH)
            def _():
                idx_copy(cur + 1, 1 - p).wait()
                for cp in gather_copies(cur + 1, 1 - p):
                    cp.start()

                @pl.when(cur + 2 < NCH)
                def _():
                    idx_copy(cur + 2, p).start()

            g_v = g_bufs[p]
            vals_v = vals_bufs[p]

            @pl.loop(0, C, unroll=2)
            def _row(r):
                base = r * NNZ
                vrow = vals_v[pl.ds(base, NNZ)]
                vs_splat = [jnp.full((LANES,), vrow[k], jnp.float32)
                            for k in range(NNZ)]
                vsb = [plsc.pack(s, s, format=plsc.PackFormat.INTERLEAVED)
                       for s in vs_splat]
                col_v = jnp.full((LANES,), cur * C + r, jnp.int32)
                for c32 in range(BATCH // 32):
                    prods = [
                        vsb[k] * g_v[base + k, pl.ds(c32 * 32, 32)]
                        for k in range(NNZ)
                    ]
                    while len(prods) > 1:
                        prods = [prods[i] + prods[i + 1]
                                 for i in range(0, len(prods), 2)]
                    ev, od = plsc.unpack(prods[0],
                                         format=plsc.PackFormat.INTERLEAVED)
                    plsc.addupdate_scatter(obuf, [ev_idx[c32], col_v], ev)
                    plsc.addupdate_scatter(obuf, [od_idx[c32], col_v], od)

    flush = pltpu.make_async_copy(obuf.at[:, pl.ds(0, ROWS_PER_W)],
                                  out.at[:, pl.ds(row0_w, ROWS_PER_W)], osem)
    flush.start()
    flush.wait()


_sc_call = pl.kernel(
    _sc_body,
    out_type=(
        jax.ShapeDtypeStruct((BATCH, N_ROWS), jnp.float32),
        jax.ShapeDtypeStruct((N_COLS, BATCH), jnp.bfloat16),
    ),
    mesh=plsc.VectorSubcoreMesh(core_axis_name="c", subcore_axis_name="s",
                                num_cores=NC, num_subcores=NS),
    scratch_types=[
        pltpu.VMEM((BATCH, PA_P), jnp.float32),
        pltpu.VMEM((BATCH, PA_P), jnp.float32),
        pltpu.VMEM((PA_W, BATCH), jnp.bfloat16),
        pltpu.VMEM((PA_W, BATCH), jnp.bfloat16),
        pltpu.VMEM((NG, GW), jnp.int32),
        pltpu.VMEM((NG, GW), jnp.int32),
        pltpu.VMEM((IPC,), jnp.float32),
        pltpu.VMEM((IPC,), jnp.float32),
        pltpu.VMEM((ROWS_PER_W,), jnp.float32),
        pltpu.VMEM((IPC, BATCH), jnp.bfloat16),
        pltpu.VMEM((IPC, BATCH), jnp.bfloat16),
        pltpu.VMEM((BATCH, RPW_P), jnp.float32),
        pltpu.SemaphoreType.DMA,
        pltpu.SemaphoreType.DMA,
        pltpu.SemaphoreType.DMA,
        pltpu.SemaphoreType.DMA,
        pltpu.SemaphoreType.DMA,
        pltpu.SemaphoreType.DMA,
        pltpu.SemaphoreType.DMA,
    ],
    compiler_params=pltpu.CompilerParams(use_tc_tiling_on_sc=False,
                                         needs_layout_passes=False),
)


def kernel(data, row_ptr, col_idx, values, bias):
    del row_ptr
    idx2 = col_idx.reshape(-1, GW)
    y, _ = _sc_call(data, idx2, values, bias)
    return y


# speedup vs baseline: 1.0080x; 1.0080x over previous
import jax
import jax.numpy as jnp
from jax import lax
from jax.experimental import pallas as pl
from jax.experimental.pallas import tpu as pltpu
from jax.experimental.pallas import tpu_sc as plsc

N_ROWS = 16384
N_COLS = 16384
NNZ = 16
BATCH = 64
LANES = 16

NC, NS = 2, 16
NW = NC * NS
ROWS_PER_W = N_ROWS // NW
C = 32
NCH = ROWS_PER_W // C
IPC = C * NNZ
GW = 128
NG = IPC // GW
IDXROWS_PER_W = ROWS_PER_W * NNZ // GW
RPW_P = ROWS_PER_W + 1

COLS_PER_TILE = N_COLS // NS
PA_W = 256
PA_P = PA_W + 1
PA_PASSES = COLS_PER_TILE // PA_W


def _sc_body(data, idx2, vals, bias, out, table,
             pa_src, pa_src2, pa_dst, pa_dst2,
             idx_a, idx_b, vals_a, vals_b, bias_v, g_a, g_b, obuf,
             gsem_a, gsem_b, isem_a, isem_b, osem, pasem_i, pasem_o):
    cid = lax.axis_index("c")
    sid = lax.axis_index("s")
    wid = sid * NC + cid
    row0_w = wid * ROWS_PER_W
    ir0_w = wid * IDXROWS_PER_W

    iota = lax.iota(jnp.int32, LANES)

    col0 = sid * COLS_PER_TILE
    pa_srcs = (pa_src, pa_src2)
    pa_dsts = (pa_dst, pa_dst2)

    def pa_in(pa, q):
        return pltpu.make_async_copy(
            data.at[:, pl.ds(col0 + pa * PA_W, PA_W)],
            pa_srcs[q].at[:, pl.ds(0, PA_W)], pasem_i)

    def pa_out(pa, q):
        return pltpu.make_async_copy(
            pa_dsts[q], table.at[pl.ds(col0 + pa * PA_W, PA_W)], pasem_o)

    pa_in(0, 0).start()
    for pa in range(PA_PASSES):
        q = pa % 2
        pa_in(pa, q).wait()
        if pa + 1 < PA_PASSES:
            pa_in(pa + 1, 1 - q).start()
        if pa >= 2:
            pa_out(pa - 2, q).wait()
        src = pa_srcs[q]
        dst = pa_dsts[q]

        @pl.loop(0, PA_W, unroll=4)
        def _col(c):
            cidx = jnp.full((LANES,), c, jnp.int32)
            quads = [
                plsc.load_gather(src, [iota + h * LANES, cidx])
                for h in range(BATCH // LANES)
            ]
            lo = plsc.pack(quads[0], quads[1],
                           format=plsc.PackFormat.INTERLEAVED)
            hi = plsc.pack(quads[2], quads[3],
                           format=plsc.PackFormat.INTERLEAVED)
            dst[c, pl.ds(0, 32)] = lo
            dst[c, pl.ds(32, 32)] = hi

        pa_out(pa, q).start()
    pa_out(PA_PASSES - 2, 0).wait()
    pa_out(PA_PASSES - 1, 1).wait()

    plsc.subcore_barrier()

    idx_bufs = (idx_a, idx_b)
    vals_bufs = (vals_a, vals_b)
    g_bufs = (g_a, g_b)
    gsems = (gsem_a, gsem_b)
    isems = (isem_a, isem_b)

    def idx_copy(g, p):
        return pltpu.make_async_copy(
            idx2.at[pl.ds(ir0_w + g * NG, NG)], idx_bufs[p], isems[p])

    def gather_copies(g, p):
        nnz0 = (row0_w + g * C) * NNZ
        cps = [
            pltpu.make_async_copy(table.at[idx_bufs[p].at[j]],
                                  g_bufs[p].at[pl.ds(j * GW, GW)], gsems[p])
            for j in range(NG)
        ]
        cps.append(pltpu.make_async_copy(vals.at[pl.ds(nnz0, IPC)],
                                         vals_bufs[p], gsems[p]))
        return cps

    pltpu.sync_copy(idx2.at[pl.ds(ir0_w, NG)], idx_bufs[0])
    for cp in gather_copies(0, 0):
        cp.start()
    idx_copy(1, 1).start()

    pltpu.sync_copy(bias.at[pl.ds(row0_w, ROWS_PER_W)], bias_v)

    @pl.loop(0, BATCH)
    def _binit(b):
        for h in range(ROWS_PER_W // LANES):
            obuf[b, pl.ds(h * LANES, LANES)] = bias_v[pl.ds(h * LANES, LANES)]

    ev_idx = [iota + c32 * 32 for c32 in range(BATCH // 32)]
    od_idx = [iota + 16 + c32 * 32 for c32 in range(BATCH // 32)]

    @pl.loop(0, NCH, step=2)
    def _pair(g0):
        for p in range(2):
            cur = g0 + p
            for cp in gather_copies(cur, p):
                cp.wait()
            @pl.when(cur + 1 < NCH)
            def _():
                idx_copy(cur + 1, 1 - p).wait()
                for cp in gather_copies(cur + 1, 1 - p):
                    cp.start()

                @pl.when(cur + 2 < NCH)
                def _():
                    idx_copy(cur + 2, p).start()

            g_v = g_bufs[p]
            vals_v = vals_bufs[p]

            @pl.loop(0, C)
            def _row(r):
                base = r * NNZ
                vrow = vals_v[pl.ds(base, NNZ)]
                vs_splat = [jnp.full((LANES,), vrow[k], jnp.float32)
                            for k in range(NNZ)]
                vsb = [plsc.pack(s, s, format=plsc.PackFormat.INTERLEAVED)
                       for s in vs_splat]
                col_v = jnp.full((LANES,), cur * C + r, jnp.int32)
                for c32 in range(BATCH // 32):
                    prods = [
                        vsb[k] * g_v[base + k, pl.ds(c32 * 32, 32)]
                        for k in range(NNZ)
                    ]
                    while len(prods) > 1:
                        prods = [prods[i] + prods[i + 1]
                                 for i in range(0, len(prods), 2)]
                    ev, od = plsc.unpack(prods[0],
                                         format=plsc.PackFormat.INTERLEAVED)
                    plsc.addupdate_scatter(obuf, [ev_idx[c32], col_v], ev)
                    plsc.addupdate_scatter(obuf, [od_idx[c32], col_v], od)

    flush = pltpu.make_async_copy(obuf.at[:, pl.ds(0, ROWS_PER_W)],
                                  out.at[:, pl.ds(row0_w, ROWS_PER_W)], osem)
    flush.start()
    flush.wait()


_sc_call = pl.kernel(
    _sc_body,
    out_type=(
        jax.ShapeDtypeStruct((BATCH, N_ROWS), jnp.float32),
        jax.ShapeDtypeStruct((N_COLS, BATCH), jnp.bfloat16),
    ),
    mesh=plsc.VectorSubcoreMesh(core_axis_name="c", subcore_axis_name="s",
                                num_cores=NC, num_subcores=NS),
    scratch_types=[
        pltpu.VMEM((BATCH, PA_P), jnp.float32),
        pltpu.VMEM((BATCH, PA_P), jnp.float32),
        pltpu.VMEM((PA_W, BATCH), jnp.bfloat16),
        pltpu.VMEM((PA_W, BATCH), jnp.bfloat16),
        pltpu.VMEM((NG, GW), jnp.int32),
        pltpu.VMEM((NG, GW), jnp.int32),
        pltpu.VMEM((IPC,), jnp.float32),
        pltpu.VMEM((IPC,), jnp.float32),
        pltpu.VMEM((ROWS_PER_W,), jnp.float32),
        pltpu.VMEM((IPC, BATCH), jnp.bfloat16),
        pltpu.VMEM((IPC, BATCH), jnp.bfloat16),
        pltpu.VMEM((BATCH, RPW_P), jnp.float32),
        pltpu.SemaphoreType.DMA,
        pltpu.SemaphoreType.DMA,
        pltpu.SemaphoreType.DMA,
        pltpu.SemaphoreType.DMA,
        pltpu.SemaphoreType.DMA,
        pltpu.SemaphoreType.DMA,
        pltpu.SemaphoreType.DMA,
    ],
    compiler_params=pltpu.CompilerParams(use_tc_tiling_on_sc=False,
                                         needs_layout_passes=False),
)


def kernel(data, row_ptr, col_idx, values, bias):
    del row_ptr
    idx2 = col_idx.reshape(-1, GW)
    y, _ = _sc_call(data, idx2, values, bias)
    return y


# speedup vs baseline: 1.0199x; 1.0119x over previous
import jax
import jax.numpy as jnp
from jax import lax
from jax.experimental import pallas as pl
from jax.experimental.pallas import tpu as pltpu
from jax.experimental.pallas import tpu_sc as plsc

N_ROWS = 16384
N_COLS = 16384
NNZ = 16
BATCH = 64
LANES = 16

NC, NS = 2, 16
NW = NC * NS
ROWS_PER_W = N_ROWS // NW
C = 32
NCH = ROWS_PER_W // C
IPC = C * NNZ
GW = 128
NG = IPC // GW
IDXROWS_PER_W = ROWS_PER_W * NNZ // GW
RPW_P = ROWS_PER_W + 1

COLS_PER_TILE = N_COLS // NS
PA_W = 256
PA_P = PA_W + 1
PA_PASSES = COLS_PER_TILE // PA_W


def _sc_body(data, idx2, vals, bias, out, table,
             pa_src, pa_src2, pa_dst, pa_dst2,
             idx_a, idx_b, vals_a, vals_b, bias_v, g_a, g_b, obuf,
             gsem_a, gsem_b, isem_a, isem_b, osem, bsem, pasem_i, pasem_o):
    cid = lax.axis_index("c")
    sid = lax.axis_index("s")
    wid = sid * NC + cid
    row0_w = wid * ROWS_PER_W
    ir0_w = wid * IDXROWS_PER_W

    iota = lax.iota(jnp.int32, LANES)

    pltpu.make_async_copy(idx2.at[pl.ds(ir0_w, NG)], idx_a, isem_a).start()
    pltpu.make_async_copy(idx2.at[pl.ds(ir0_w + NG, NG)], idx_b,
                          isem_b).start()
    pltpu.make_async_copy(bias.at[pl.ds(row0_w, ROWS_PER_W)], bias_v,
                          bsem).start()

    col0 = sid * COLS_PER_TILE
    pa_srcs = (pa_src, pa_src2)
    pa_dsts = (pa_dst, pa_dst2)

    def pa_in(pa, q):
        return pltpu.make_async_copy(
            data.at[:, pl.ds(col0 + pa * PA_W, PA_W)],
            pa_srcs[q].at[:, pl.ds(0, PA_W)], pasem_i)

    def pa_out(pa, q):
        return pltpu.make_async_copy(
            pa_dsts[q], table.at[pl.ds(col0 + pa * PA_W, PA_W)], pasem_o)

    pa_in(0, 0).start()
    for pa in range(PA_PASSES):
        q = pa % 2
        pa_in(pa, q).wait()
        if pa + 1 < PA_PASSES:
            pa_in(pa + 1, 1 - q).start()
        if pa >= 2:
            pa_out(pa - 2, q).wait()
        src = pa_srcs[q]
        dst = pa_dsts[q]

        @pl.loop(0, PA_W, unroll=4)
        def _col(c):
            cidx = jnp.full((LANES,), c, jnp.int32)
            quads = [
                plsc.load_gather(src, [iota + h * LANES, cidx])
                for h in range(BATCH // LANES)
            ]
            lo = plsc.pack(quads[0], quads[1],
                           format=plsc.PackFormat.INTERLEAVED)
            hi = plsc.pack(quads[2], quads[3],
                           format=plsc.PackFormat.INTERLEAVED)
            dst[c, pl.ds(0, 32)] = lo
            dst[c, pl.ds(32, 32)] = hi

        pa_out(pa, q).start()
    pa_out(PA_PASSES - 2, 0).wait()
    pa_out(PA_PASSES - 1, 1).wait()

    plsc.subcore_barrier()

    idx_bufs = (idx_a, idx_b)
    vals_bufs = (vals_a, vals_b)
    g_bufs = (g_a, g_b)
    gsems = (gsem_a, gsem_b)
    isems = (isem_a, isem_b)

    def idx_copy(g, p):
        return pltpu.make_async_copy(
            idx2.at[pl.ds(ir0_w + g * NG, NG)], idx_bufs[p], isems[p])

    def gather_copies(g, p):
        nnz0 = (row0_w + g * C) * NNZ
        cps = [
            pltpu.make_async_copy(table.at[idx_bufs[p].at[j]],
                                  g_bufs[p].at[pl.ds(j * GW, GW)], gsems[p])
            for j in range(NG)
        ]
        cps.append(pltpu.make_async_copy(vals.at[pl.ds(nnz0, IPC)],
                                         vals_bufs[p], gsems[p]))
        return cps

    pltpu.make_async_copy(idx2.at[pl.ds(ir0_w, NG)], idx_a, isem_a).wait()
    for cp in gather_copies(0, 0):
        cp.start()

    pltpu.make_async_copy(bias.at[pl.ds(row0_w, ROWS_PER_W)], bias_v,
                          bsem).wait()

    @pl.loop(0, BATCH)
    def _binit(b):
        for h in range(ROWS_PER_W // LANES):
            obuf[b, pl.ds(h * LANES, LANES)] = bias_v[pl.ds(h * LANES, LANES)]

    ev_idx = [iota + c32 * 32 for c32 in range(BATCH // 32)]
    od_idx = [iota + 16 + c32 * 32 for c32 in range(BATCH // 32)]

    HALF = ROWS_PER_W // 2

    def half_flush(h):
        return pltpu.make_async_copy(
            obuf.at[:, pl.ds(h * HALF, HALF)],
            out.at[:, pl.ds(row0_w + h * HALF, HALF)], osem)

    @pl.loop(0, NCH, step=2)
    def _pair(g0):
        for p in range(2):
            cur = g0 + p
            for cp in gather_copies(cur, p):
                cp.wait()
            @pl.when(cur + 1 < NCH)
            def _():
                idx_copy(cur + 1, 1 - p).wait()
                for cp in gather_copies(cur + 1, 1 - p):
                    cp.start()

                @pl.when(cur + 2 < NCH)
                def _():
                    idx_copy(cur + 2, p).start()

            g_v = g_bufs[p]
            vals_v = vals_bufs[p]

            @pl.loop(0, C)
            def _row(r):
                base = r * NNZ
                vrow = vals_v[pl.ds(base, NNZ)]
                vs_splat = [jnp.full((LANES,), vrow[k], jnp.float32)
                            for k in range(NNZ)]
                vsb = [plsc.pack(s, s, format=plsc.PackFormat.INTERLEAVED)
                       for s in vs_splat]
                col_v = jnp.full((LANES,), cur * C + r, jnp.int32)
                for c32 in range(BATCH // 32):
                    prods = [
                        vsb[k] * g_v[base + k, pl.ds(c32 * 32, 32)]
                        for k in range(NNZ)
                    ]
                    while len(prods) > 1:
                        prods = [prods[i] + prods[i + 1]
                                 for i in range(0, len(prods), 2)]
                    ev, od = plsc.unpack(prods[0],
                                         format=plsc.PackFormat.INTERLEAVED)
                    plsc.addupdate_scatter(obuf, [ev_idx[c32], col_v], ev)
                    plsc.addupdate_scatter(obuf, [od_idx[c32], col_v], od)

            @pl.when(cur == NCH // 2 - 1)
            def _():
                half_flush(0).start()

    half_flush(1).start()
    half_flush(0).wait()
    half_flush(1).wait()


_sc_call = pl.kernel(
    _sc_body,
    out_type=(
        jax.ShapeDtypeStruct((BATCH, N_ROWS), jnp.float32),
        jax.ShapeDtypeStruct((N_COLS, BATCH), jnp.bfloat16),
    ),
    mesh=plsc.VectorSubcoreMesh(core_axis_name="c", subcore_axis_name="s",
                                num_cores=NC, num_subcores=NS),
    scratch_types=[
        pltpu.VMEM((BATCH, PA_P), jnp.float32),
        pltpu.VMEM((BATCH, PA_P), jnp.float32),
        pltpu.VMEM((PA_W, BATCH), jnp.bfloat16),
        pltpu.VMEM((PA_W, BATCH), jnp.bfloat16),
        pltpu.VMEM((NG, GW), jnp.int32),
        pltpu.VMEM((NG, GW), jnp.int32),
        pltpu.VMEM((IPC,), jnp.float32),
        pltpu.VMEM((IPC,), jnp.float32),
        pltpu.VMEM((ROWS_PER_W,), jnp.float32),
        pltpu.VMEM((IPC, BATCH), jnp.bfloat16),
        pltpu.VMEM((IPC, BATCH), jnp.bfloat16),
        pltpu.VMEM((BATCH, RPW_P), jnp.float32),
        pltpu.SemaphoreType.DMA,
        pltpu.SemaphoreType.DMA,
        pltpu.SemaphoreType.DMA,
        pltpu.SemaphoreType.DMA,
        pltpu.SemaphoreType.DMA,
        pltpu.SemaphoreType.DMA,
        pltpu.SemaphoreType.DMA,
        pltpu.SemaphoreType.DMA,
    ],
    compiler_params=pltpu.CompilerParams(use_tc_tiling_on_sc=False,
                                         needs_layout_passes=False),
)


def kernel(data, row_ptr, col_idx, values, bias):
    del row_ptr
    idx2 = col_idx.reshape(-1, GW)
    y, _ = _sc_call(data, idx2, values, bias)
    return y
